# Initial kernel scaffold; baseline (speedup 1.0000x reference)
import jax
import jax.numpy as jnp
from jax.experimental import pallas as pl


def kernel(input, target):
    raise NotImplementedError("write your pallas kernel here")


# speedup vs baseline: 20.9283x; 20.9283x over previous
import jax
import jax.numpy as jnp
import numpy as np
from jax import lax
from jax.experimental import pallas as pl
from jax.experimental.pallas import tpu as pltpu

_R, _C = 4096, 1024
_N = _R * _C
_CHUNK = 256
_NCH = _R // _CHUNK
_INT_MIN = np.int32(-(2 ** 31))


def _log1p_exp_neg_abs(x):
    return jnp.log1p(jnp.exp(-jnp.abs(x)))


def _ohnm_body(x_ref, t_ref, o_ref, key_ref):
    def stats_chunk(i, carry):
        pos_cnt, pos_loss = carry
        x = x_ref[pl.ds(i * _CHUNK, _CHUNK), :]
        t = t_ref[pl.ds(i * _CHUNK, _CHUNK), :]
        pos = t > 0.0
        pos_cnt = pos_cnt + jnp.sum(pos.astype(jnp.int32))
        bce = jnp.maximum(x, 0.0) - x * t + _log1p_exp_neg_abs(x)
        pos_loss = pos_loss + jnp.sum(jnp.where(pos, bce, 0.0))
        b = lax.bitcast_convert_type(x, jnp.int32)
        key = jnp.where(b >= 0, b, jnp.bitwise_xor(jnp.bitwise_not(b), _INT_MIN))
        key = jnp.where(pos, _INT_MIN, key)
        key_ref[pl.ds(i * _CHUNK, _CHUNK), :] = key
        return pos_cnt, pos_loss

    pos_cnt, pos_loss = lax.fori_loop(
        0, _NCH, stats_chunk, (jnp.int32(0), jnp.float32(0.0))
    )

    neg_cnt = jnp.int32(_N) - pos_cnt
    k = (pos_cnt.astype(jnp.float32) * 3.0).astype(jnp.int32)
    k_eff = jnp.minimum(k, neg_cnt)

    def count_ge(cand):
        def cchunk(i, c):
            s = key_ref[pl.ds(i * _CHUNK, _CHUNK), :]
            return c + jnp.sum((s >= cand).astype(jnp.int32))

        return lax.fori_loop(0, _NCH, cchunk, jnp.int32(0))

    def bit_step(i, thr_b):
        cand_b = jnp.bitwise_or(thr_b, jnp.left_shift(jnp.int32(1), 31 - i))
        cand = jnp.bitwise_xor(cand_b, _INT_MIN)
        cnt = count_ge(cand)
        return jnp.where(cnt >= k_eff, cand_b, thr_b)

    thr_b = lax.fori_loop(0, 32, bit_step, jnp.int32(0))
    thr = jnp.bitwise_xor(thr_b, _INT_MIN)

    def final_chunk(i, carry):
        cnt_gt, sum_gt, cnt_eq, sum_eq = carry
        s = key_ref[pl.ds(i * _CHUNK, _CHUNK), :]
        x = x_ref[pl.ds(i * _CHUNK, _CHUNK), :]
        sp = jnp.maximum(x, 0.0) + _log1p_exp_neg_abs(x)
        gt = s > thr
        eq = s == thr
        cnt_gt = cnt_gt + jnp.sum(gt.astype(jnp.int32))
        sum_gt = sum_gt + jnp.sum(jnp.where(gt, sp, 0.0))
        cnt_eq = cnt_eq + jnp.sum(eq.astype(jnp.int32))
        sum_eq = sum_eq + jnp.sum(jnp.where(eq, sp, 0.0))
        return cnt_gt, sum_gt, cnt_eq, sum_eq

    cnt_gt, sum_gt, cnt_eq, sum_eq = lax.fori_loop(
        0, _NCH, final_chunk,
        (jnp.int32(0), jnp.float32(0.0), jnp.int32(0), jnp.float32(0.0)),
    )

    tie_cnt = k_eff - cnt_gt
    sp_thr = jnp.where(cnt_eq > 0, sum_eq / cnt_eq.astype(jnp.float32), 0.0)
    tie = jnp.where(tie_cnt > 0, tie_cnt.astype(jnp.float32) * sp_thr, 0.0)

    total = pos_loss + sum_gt + tie
    denom = (pos_cnt + k).astype(jnp.float32)
    o_ref[0, 0] = total / denom


def kernel(input, target):
    x = input.reshape(_R, _C)
    t = target.reshape(_R, _C)
    out = pl.pallas_call(
        _ohnm_body,
        out_shape=jax.ShapeDtypeStruct((1, 1), jnp.float32),
        in_specs=[
            pl.BlockSpec((_R, _C), lambda: (0, 0)),
            pl.BlockSpec((_R, _C), lambda: (0, 0)),
        ],
        out_specs=pl.BlockSpec(memory_space=pltpu.SMEM),
        scratch_shapes=[pltpu.VMEM((_R, _C), jnp.int32)],
    )(x, t)
    return out[0, 0]


# speedup vs baseline: 24.9682x; 1.1930x over previous
import jax
import jax.numpy as jnp
import numpy as np
from jax import lax
from jax.experimental import pallas as pl
from jax.experimental.pallas import tpu as pltpu

_R, _C = 4096, 1024
_N = _R * _C
_CHUNK = 256
_NCH = _R // _CHUNK
_INT_MIN = np.int32(-(2 ** 31))


def _log1p_exp_neg_abs(x):
    return jnp.log1p(jnp.exp(-jnp.abs(x)))


def _ohnm_body(x_ref, t_ref, o_ref, key_ref):
    def stats_chunk(i, carry):
        pos_cnt, pos_loss = carry
        x = x_ref[pl.ds(i * _CHUNK, _CHUNK), :]
        t = t_ref[pl.ds(i * _CHUNK, _CHUNK), :]
        pos = t > 0.0
        pos_cnt = pos_cnt + jnp.sum(pos.astype(jnp.int32))
        bce = jnp.maximum(x, 0.0) - x * t + _log1p_exp_neg_abs(x)
        pos_loss = pos_loss + jnp.sum(jnp.where(pos, bce, 0.0))
        b = lax.bitcast_convert_type(x, jnp.int32)
        key = jnp.where(b >= 0, b, jnp.bitwise_xor(jnp.bitwise_not(b), _INT_MIN))
        key = jnp.where(pos, _INT_MIN, key)
        key_ref[pl.ds(i * _CHUNK, _CHUNK), :] = key
        return pos_cnt, pos_loss

    pos_cnt, pos_loss = lax.fori_loop(
        0, _NCH, stats_chunk, (jnp.int32(0), jnp.float32(0.0))
    )

    neg_cnt = jnp.int32(_N) - pos_cnt
    k = (pos_cnt.astype(jnp.float32) * 3.0).astype(jnp.int32)
    k_eff = jnp.minimum(k, neg_cnt)

    def count_ge(cand):
        def cchunk(i, c):
            s = key_ref[pl.ds(i * _CHUNK, _CHUNK), :]
            return c + jnp.sum((s >= cand).astype(jnp.int32))

        return lax.fori_loop(0, _NCH, cchunk, jnp.int32(0))

    def bit_step(i, thr_b):
        cand_b = jnp.bitwise_or(thr_b, jnp.left_shift(jnp.int32(1), 31 - i))
        cand = jnp.bitwise_xor(cand_b, _INT_MIN)
        cnt = count_ge(cand)
        return jnp.where(cnt >= k_eff, cand_b, thr_b)

    thr_b = lax.fori_loop(0, 24, bit_step, jnp.int32(0))
    thr = jnp.bitwise_xor(thr_b, _INT_MIN)

    def final_chunk(i, carry):
        cnt_gt, sum_gt, cnt_eq, sum_eq = carry
        s = key_ref[pl.ds(i * _CHUNK, _CHUNK), :]
        x = x_ref[pl.ds(i * _CHUNK, _CHUNK), :]
        sp = jnp.maximum(x, 0.0) + _log1p_exp_neg_abs(x)
        s_pref = jnp.bitwise_and(s, np.int32(-256))
        gt = s_pref > thr
        eq = s_pref == thr
        cnt_gt = cnt_gt + jnp.sum(gt.astype(jnp.int32))
        sum_gt = sum_gt + jnp.sum(jnp.where(gt, sp, 0.0))
        cnt_eq = cnt_eq + jnp.sum(eq.astype(jnp.int32))
        sum_eq = sum_eq + jnp.sum(jnp.where(eq, sp, 0.0))
        return cnt_gt, sum_gt, cnt_eq, sum_eq

    cnt_gt, sum_gt, cnt_eq, sum_eq = lax.fori_loop(
        0, _NCH, final_chunk,
        (jnp.int32(0), jnp.float32(0.0), jnp.int32(0), jnp.float32(0.0)),
    )

    tie_cnt = k_eff - cnt_gt
    sp_thr = jnp.where(cnt_eq > 0, sum_eq / cnt_eq.astype(jnp.float32), 0.0)
    tie = jnp.where(tie_cnt > 0, tie_cnt.astype(jnp.float32) * sp_thr, 0.0)

    total = pos_loss + sum_gt + tie
    denom = (pos_cnt + k).astype(jnp.float32)
    o_ref[0, 0] = total / denom


def kernel(input, target):
    x = input.reshape(_R, _C)
    t = target.reshape(_R, _C)
    out = pl.pallas_call(
        _ohnm_body,
        out_shape=jax.ShapeDtypeStruct((1, 1), jnp.float32),
        in_specs=[
            pl.BlockSpec((_R, _C), lambda: (0, 0)),
            pl.BlockSpec((_R, _C), lambda: (0, 0)),
        ],
        out_specs=pl.BlockSpec(memory_space=pltpu.SMEM),
        scratch_shapes=[pltpu.VMEM((_R, _C), jnp.int32)],
    )(x, t)
    return out[0, 0]
